# baseline (speedup 1.0000x reference)
import functools

import jax
import jax.numpy as jnp
from jax import lax
from jax.experimental import pallas as pl
from jax.experimental.pallas import tpu as pltpu
from jax.experimental.pallas import tpu_sc as plsc

N = 10000
E = 320000
D = 128
NPAD = 10240
STRIPE = NPAD // 16
NTILES = 32
EPT = E // NTILES
C = 112
EPTP = 10080
NCH = EPTP // C

f32 = jnp.float32
_mesh = plsc.VectorSubcoreMesh(core_axis_name="c", subcore_axis_name="s")


def _make_seg_kernel(with_q):
  out_type = [
      jax.ShapeDtypeStruct((2, NPAD, D), f32),
      jax.ShapeDtypeStruct((2, NPAD), f32),
  ]
  scratch_types = (
      [pltpu.VMEM((NCH, C), jnp.int32),
       pltpu.VMEM((NCH, C), jnp.int32),
       pltpu.VMEM((C, D), f32),
       pltpu.VMEM((C,), f32),
       pltpu.VMEM_SHARED((NPAD, D), f32),
       pltpu.VMEM_SHARED((NPAD,), f32)] +
      [pltpu.SemaphoreType.DMA for _ in range(2)]
  )

  def body(src_h, dst_h, table_h, w_h, zrows_h, z1_h, ones_h, *refs):
    (rows_out, sca_out, srcv, dstv, rows, vals,
     acc_sh, sca_sh, sem_g, sem_gv) = refs

    c = lax.axis_index("c")
    s = lax.axis_index("s")
    wid = c * 16 + s
    r0 = s * STRIPE
    pltpu.sync_copy(zrows_h, acc_sh.at[pl.ds(r0, STRIPE)])
    pltpu.sync_copy(z1_h, sca_sh.at[pl.ds(r0, STRIPE)])
    pltpu.sync_copy(src_h.at[wid], srcv)
    pltpu.sync_copy(dst_h.at[wid], dstv)
    if not with_q:
      pltpu.sync_copy(ones_h, vals)
    plsc.subcore_barrier()

    def chunk(j, carry):
      g = pltpu.async_copy(table_h.at[srcv.at[j]], rows, sem_g)
      if with_q:
        gv = pltpu.async_copy(w_h.at[dstv.at[j]], vals, sem_gv)
      g.wait()
      pltpu.sync_copy(rows, acc_sh.at[dstv.at[j]], add=True)
      if with_q:
        gv.wait()
        pltpu.sync_copy(vals, sca_sh.at[srcv.at[j]], add=True)
      else:
        pltpu.sync_copy(vals, sca_sh.at[dstv.at[j]], add=True)
      return carry

    lax.fori_loop(0, NCH, chunk, 0)
    plsc.subcore_barrier()
    pltpu.sync_copy(acc_sh.at[pl.ds(r0, STRIPE)],
                    rows_out.at[c, pl.ds(r0, STRIPE)])
    pltpu.sync_copy(sca_sh.at[pl.ds(r0, STRIPE)],
                    sca_out.at[c, pl.ds(r0, STRIPE)])

  return pl.kernel(body, mesh=_mesh, out_type=out_type,
                   scratch_types=scratch_types)


_seg_cnt = _make_seg_kernel(with_q=False)
_seg_q = _make_seg_kernel(with_q=True)


BLK = 2000


def _dot_t(a, b):
  return lax.dot_general(a, b, (((1,), (1,)), ((), ())),
                         preferred_element_type=f32)


def _tc1_body(p0, p1, c0, c1, x, wl, b, wr, h_out, inv_out):
  cnt = c0[...] + c1[...]
  inv = 1.0 / jnp.maximum(cnt, 1.0)
  mean = (p0[...] + p1[...]) * inv
  h = _dot_t(mean, wl[...]) + b[...] + _dot_t(x[...], wr[...])
  h_out[...] = jnp.maximum(h, 0.0)
  inv_out[...] = inv


def _tc1(p0, p1, c0, c1, x, wl, b, wr):
  nb = N // BLK
  row = pl.BlockSpec((BLK, D), lambda i: (i, 0))
  col = pl.BlockSpec((BLK, 1), lambda i: (i, 0))
  full = pl.BlockSpec((D, D), lambda i: (0, 0))
  bias = pl.BlockSpec((1, D), lambda i: (0, 0))
  return pl.pallas_call(
      _tc1_body,
      grid=(nb,),
      in_specs=[row, row, col, col, row, full, bias, full],
      out_specs=[row, col],
      out_shape=[jax.ShapeDtypeStruct((N, D), f32),
                 jax.ShapeDtypeStruct((N, 1), f32)],
  )(p0, p1, c0, c1, x, wl, b, wr)


def _tc2_body(p0, p1, inv, h1, q0, q1, w2l, b2, w2r, w3l, b3, w3r,
              out, s1_acc, s2_acc):
  i = pl.program_id(0)
  mean = (p0[...] + p1[...]) * inv[...]
  h2 = jnp.maximum(_dot_t(mean, w2l[...]) + b2[...] + _dot_t(h1[...], w2r[...]),
                   0.0)
  q = q0[...] + q1[...]

  @pl.when(i == 0)
  def _():
    s1_acc[...] = jnp.zeros_like(s1_acc)
    s2_acc[...] = jnp.zeros_like(s2_acc)

  s1_acc[...] += jnp.sum(q * h2, axis=0, keepdims=True)
  s2_acc[...] += jnp.sum(h2, axis=0, keepdims=True)

  @pl.when(i == pl.num_programs(0) - 1)
  def _():
    out[...] = ((_dot_t(s1_acc[...], w3l[...])
                 + _dot_t(s2_acc[...], w3r[...])) * (1.0 / N) + b3[...])


def _tc2(p0, p1, inv, h1, q0, q1, w2l, b2, w2r, w3l, b3, w3r):
  nb = N // BLK
  row = pl.BlockSpec((BLK, D), lambda i: (i, 0))
  col = pl.BlockSpec((BLK, 1), lambda i: (i, 0))
  full = pl.BlockSpec((D, D), lambda i: (0, 0))
  bias = pl.BlockSpec((1, D), lambda i: (0, 0))
  return pl.pallas_call(
      _tc2_body,
      grid=(nb,),
      in_specs=[row, row, col, row, col, col,
                full, bias, full, full, bias, full],
      out_specs=pl.BlockSpec((1, D), lambda i: (0, 0)),
      out_shape=jax.ShapeDtypeStruct((1, D), f32),
      scratch_shapes=[pltpu.VMEM((1, D), f32), pltpu.VMEM((1, D), f32)],
      compiler_params=pltpu.CompilerParams(
          dimension_semantics=("arbitrary",)),
  )(p0, p1, inv, h1, q0, q1, w2l, b2, w2r, w3l, b3, w3r)


def kernel(x, edge_index, W1l, b1, W1r, W2l, b2, W2r, W3l, b3, W3r):
  ei = edge_index.astype(jnp.int32)
  npad_e = EPTP - EPT
  src = jnp.concatenate(
      [ei[0].reshape(NTILES, EPT),
       jnp.zeros((NTILES, npad_e), jnp.int32)], axis=1).reshape(NTILES, NCH, C)
  dst = jnp.concatenate(
      [ei[1].reshape(NTILES, EPT),
       jnp.full((NTILES, npad_e), NPAD - 1, jnp.int32)],
      axis=1).reshape(NTILES, NCH, C)
  zrows = jnp.zeros((STRIPE, D), f32)
  z1 = jnp.zeros((STRIPE,), f32)
  ones = jnp.ones((C,), f32)
  wdummy = jnp.zeros((NPAD,), f32)

  p_l1, cnt_p = _seg_cnt(src, dst, x, wdummy, zrows, z1, ones)
  h1, inv = _tc1(p_l1[0, :N], p_l1[1, :N],
                 cnt_p[0, :N, None], cnt_p[1, :N, None],
                 x, W1l, b1[None, :], W1r)
  invpad = jnp.concatenate([inv[:, 0], jnp.zeros((NPAD - N,), f32)])
  p_l2, q_p = _seg_q(src, dst, h1, invpad, zrows, z1, ones)
  out = _tc2(p_l2[0, :N], p_l2[1, :N], inv, h1,
             q_p[0, :N, None], q_p[1, :N, None],
             W2l, b2[None, :], W2r, W3l, b3[None, :], W3r)
  return out


# speedup vs baseline: 1.2356x; 1.2356x over previous
import functools

import jax
import jax.numpy as jnp
from jax import lax
from jax.experimental import pallas as pl
from jax.experimental.pallas import tpu as pltpu
from jax.experimental.pallas import tpu_sc as plsc

N = 10000
E = 320000
D = 128
NPAD = 10240
STRIPE = NPAD // 16
NTILES = 32
EPT = E // NTILES
C = 80
EPTP = 10000
NCH = EPTP // C

f32 = jnp.float32
_mesh = plsc.VectorSubcoreMesh(core_axis_name="c", subcore_axis_name="s")


def _make_seg_kernel(with_q):
  out_type = [
      jax.ShapeDtypeStruct((2, NPAD, D), f32),
      jax.ShapeDtypeStruct((2, NPAD), f32),
  ]
  scratch_types = (
      [pltpu.VMEM((NCH, C), jnp.int32),
       pltpu.VMEM((NCH, C), jnp.int32),
       pltpu.VMEM((C, D), f32),
       pltpu.VMEM((C,), f32),
       pltpu.VMEM_SHARED((NPAD, D), f32),
       pltpu.VMEM_SHARED((NPAD,), f32)] +
      [pltpu.SemaphoreType.DMA for _ in range(2)]
  )

  def body(src_h, dst_h, table_h, w_h, zrows_h, z1_h, ones_h, *refs):
    (rows_out, sca_out, srcv, dstv, rows, vals,
     acc_sh, sca_sh, sem_g, sem_gv) = refs

    c = lax.axis_index("c")
    s = lax.axis_index("s")
    wid = c * 16 + s
    r0 = s * STRIPE
    pltpu.sync_copy(zrows_h, acc_sh.at[pl.ds(r0, STRIPE)])
    pltpu.sync_copy(z1_h, sca_sh.at[pl.ds(r0, STRIPE)])
    pltpu.sync_copy(src_h.at[wid], srcv)
    pltpu.sync_copy(dst_h.at[wid], dstv)
    if not with_q:
      pltpu.sync_copy(ones_h, vals)
    plsc.subcore_barrier()

    def chunk(j, carry):
      g = pltpu.async_copy(table_h.at[srcv.at[j]], rows, sem_g)
      if with_q:
        gv = pltpu.async_copy(w_h.at[dstv.at[j]], vals, sem_gv)
      g.wait()
      pltpu.sync_copy(rows, acc_sh.at[dstv.at[j]], add=True)
      if with_q:
        gv.wait()
        pltpu.sync_copy(vals, sca_sh.at[srcv.at[j]], add=True)
      else:
        pltpu.sync_copy(vals, sca_sh.at[dstv.at[j]], add=True)
      return carry

    lax.fori_loop(0, NCH, chunk, 0)
    plsc.subcore_barrier()
    pltpu.sync_copy(acc_sh.at[pl.ds(r0, STRIPE)],
                    rows_out.at[c, pl.ds(r0, STRIPE)])
    pltpu.sync_copy(sca_sh.at[pl.ds(r0, STRIPE)],
                    sca_out.at[c, pl.ds(r0, STRIPE)])

  return pl.kernel(body, mesh=_mesh, out_type=out_type,
                   scratch_types=scratch_types)


_seg_cnt = _make_seg_kernel(with_q=False)
_seg_q = _make_seg_kernel(with_q=True)


BLK = 2000


def _dot_t(a, b):
  return lax.dot_general(a, b, (((1,), (1,)), ((), ())),
                         preferred_element_type=f32)


def _tc1_body(p0, p1, c0, c1, x, wl, b, wr, h_out, inv_out):
  cnt = c0[...] + c1[...]
  inv = 1.0 / jnp.maximum(cnt, 1.0)
  mean = (p0[...] + p1[...]) * inv
  h = _dot_t(mean, wl[...]) + b[...] + _dot_t(x[...], wr[...])
  h_out[...] = jnp.maximum(h, 0.0)
  inv_out[...] = inv


def _tc1(p0, p1, c0, c1, x, wl, b, wr):
  nb = N // BLK
  row = pl.BlockSpec((BLK, D), lambda i: (i, 0))
  col = pl.BlockSpec((BLK, 1), lambda i: (i, 0))
  full = pl.BlockSpec((D, D), lambda i: (0, 0))
  bias = pl.BlockSpec((1, D), lambda i: (0, 0))
  return pl.pallas_call(
      _tc1_body,
      grid=(nb,),
      in_specs=[row, row, col, col, row, full, bias, full],
      out_specs=[row, col],
      out_shape=[jax.ShapeDtypeStruct((N, D), f32),
                 jax.ShapeDtypeStruct((N, 1), f32)],
  )(p0, p1, c0, c1, x, wl, b, wr)


def _tc2_body(p0, p1, inv, h1, q0, q1, w2l, b2, w2r, w3l, b3, w3r,
              out, s1_acc, s2_acc):
  i = pl.program_id(0)
  mean = (p0[...] + p1[...]) * inv[...]
  h2 = jnp.maximum(_dot_t(mean, w2l[...]) + b2[...] + _dot_t(h1[...], w2r[...]),
                   0.0)
  q = q0[...] + q1[...]

  @pl.when(i == 0)
  def _():
    s1_acc[...] = jnp.zeros_like(s1_acc)
    s2_acc[...] = jnp.zeros_like(s2_acc)

  s1_acc[...] += jnp.sum(q * h2, axis=0, keepdims=True)
  s2_acc[...] += jnp.sum(h2, axis=0, keepdims=True)

  @pl.when(i == pl.num_programs(0) - 1)
  def _():
    out[...] = ((_dot_t(s1_acc[...], w3l[...])
                 + _dot_t(s2_acc[...], w3r[...])) * (1.0 / N) + b3[...])


def _tc2(p0, p1, inv, h1, q0, q1, w2l, b2, w2r, w3l, b3, w3r):
  nb = N // BLK
  row = pl.BlockSpec((BLK, D), lambda i: (i, 0))
  col = pl.BlockSpec((BLK, 1), lambda i: (i, 0))
  full = pl.BlockSpec((D, D), lambda i: (0, 0))
  bias = pl.BlockSpec((1, D), lambda i: (0, 0))
  return pl.pallas_call(
      _tc2_body,
      grid=(nb,),
      in_specs=[row, row, col, row, col, col,
                full, bias, full, full, bias, full],
      out_specs=pl.BlockSpec((1, D), lambda i: (0, 0)),
      out_shape=jax.ShapeDtypeStruct((1, D), f32),
      scratch_shapes=[pltpu.VMEM((1, D), f32), pltpu.VMEM((1, D), f32)],
      compiler_params=pltpu.CompilerParams(
          dimension_semantics=("arbitrary",)),
  )(p0, p1, inv, h1, q0, q1, w2l, b2, w2r, w3l, b3, w3r)


def kernel(x, edge_index, W1l, b1, W1r, W2l, b2, W2r, W3l, b3, W3r):
  ei = edge_index.astype(jnp.int32)
  npad_e = EPTP - EPT
  src = jnp.concatenate(
      [ei[0].reshape(NTILES, EPT),
       jnp.zeros((NTILES, npad_e), jnp.int32)], axis=1).reshape(NTILES, NCH, C)
  dst = jnp.concatenate(
      [ei[1].reshape(NTILES, EPT),
       jnp.full((NTILES, npad_e), NPAD - 1, jnp.int32)],
      axis=1).reshape(NTILES, NCH, C)
  zrows = jnp.zeros((STRIPE, D), f32)
  z1 = jnp.zeros((STRIPE,), f32)
  ones = jnp.ones((C,), f32)
  wdummy = jnp.zeros((NPAD,), f32)

  p_l1, cnt_p = _seg_cnt(src, dst, x, wdummy, zrows, z1, ones)
  h1, inv = _tc1(p_l1[0, :N], p_l1[1, :N],
                 cnt_p[0, :N, None], cnt_p[1, :N, None],
                 x, W1l, b1[None, :], W1r)
  invpad = jnp.concatenate([inv[:, 0], jnp.zeros((NPAD - N,), f32)])
  p_l2, q_p = _seg_q(src, dst, h1, invpad, zrows, z1, ones)
  out = _tc2(p_l2[0, :N], p_l2[1, :N], inv, h1,
             q_p[0, :N, None], q_p[1, :N, None],
             W2l, b2[None, :], W2r, W3l, b3[None, :], W3r)
  return out


# speedup vs baseline: 1.2699x; 1.0278x over previous
import functools

import jax
import jax.numpy as jnp
from jax import lax
from jax.experimental import pallas as pl
from jax.experimental.pallas import tpu as pltpu
from jax.experimental.pallas import tpu_sc as plsc

N = 10000
E = 320000
D = 128
NPAD = 10240
STRIPE = NPAD // 16
NTILES = 32
EPT = E // NTILES
C = 80
EPTP = 10000
NCH = EPTP // C

f32 = jnp.float32
_mesh = plsc.VectorSubcoreMesh(core_axis_name="c", subcore_axis_name="s")


def _make_seg_kernel(with_q):
  out_type = [
      jax.ShapeDtypeStruct((2, NPAD, D), f32),
      jax.ShapeDtypeStruct((2, NPAD), f32),
  ]
  scratch_types = (
      [pltpu.VMEM((NCH, C), jnp.int32),
       pltpu.VMEM((NCH, C), jnp.int32),
       pltpu.VMEM((C, D), f32),
       pltpu.VMEM((C,), f32),
       pltpu.VMEM_SHARED((NPAD, D), f32),
       pltpu.VMEM_SHARED((NPAD,), f32)] +
      [pltpu.SemaphoreType.DMA for _ in range(2)]
  )

  def body(src_h, dst_h, table_h, w_h, zrows_h, z1_h, ones_h, *refs):
    (rows_out, sca_out, srcv, dstv, rows, vals,
     acc_sh, sca_sh, sem_g, sem_gv) = refs

    c = lax.axis_index("c")
    s = lax.axis_index("s")
    wid = c * 16 + s
    r0 = s * STRIPE
    pltpu.sync_copy(zrows_h, acc_sh.at[pl.ds(r0, STRIPE)])
    pltpu.sync_copy(z1_h, sca_sh.at[pl.ds(r0, STRIPE)])
    pltpu.sync_copy(src_h.at[wid], srcv)
    pltpu.sync_copy(dst_h.at[wid], dstv)
    if not with_q:
      pltpu.sync_copy(ones_h, vals)
    plsc.subcore_barrier()

    def chunk(j, carry):
      g = pltpu.async_copy(table_h.at[srcv.at[j]], rows, sem_g)
      if with_q:
        gv = pltpu.async_copy(w_h.at[dstv.at[j]], vals, sem_gv)
      g.wait()
      pltpu.sync_copy(rows, acc_sh.at[dstv.at[j]], add=True)
      if with_q:
        gv.wait()
        pltpu.sync_copy(vals, sca_sh.at[srcv.at[j]], add=True)
      else:
        pltpu.sync_copy(vals, sca_sh.at[dstv.at[j]], add=True)
      return carry

    lax.fori_loop(0, NCH, chunk, 0)
    plsc.subcore_barrier()
    pltpu.sync_copy(acc_sh.at[pl.ds(r0, STRIPE)],
                    rows_out.at[c, pl.ds(r0, STRIPE)])
    pltpu.sync_copy(sca_sh.at[pl.ds(r0, STRIPE)],
                    sca_out.at[c, pl.ds(r0, STRIPE)])

  return pl.kernel(body, mesh=_mesh, out_type=out_type,
                   scratch_types=scratch_types)


_seg_cnt = _make_seg_kernel(with_q=False)
_seg_q = _make_seg_kernel(with_q=True)


BLK = 2048
NB = NPAD // BLK


def _dot_t(a, b):
  return lax.dot_general(a, b, (((1,), (1,)), ((), ())),
                         preferred_element_type=f32)


def _row_mask(i):
  ridx = i * BLK + lax.broadcasted_iota(jnp.int32, (BLK, 1), 0)
  return ridx < N


def _tc1_body(p, cc, x, wl, b, wr, h_out, inv_out):
  cnt = cc[0, :, :] + cc[1, :, :]
  inv = 1.0 / jnp.maximum(cnt, 1.0)
  mean = (p[0, :, :] + p[1, :, :]) * inv
  h = _dot_t(mean, wl[...]) + b[...] + _dot_t(x[...], wr[...])
  h_out[...] = jnp.maximum(h, 0.0)
  inv_out[...] = jnp.where(_row_mask(pl.program_id(0)), inv, 0.0)


def _tc1(p, cc, x, wl, b, wr):
  row3 = pl.BlockSpec((2, BLK, D), lambda i: (0, i, 0))
  col3 = pl.BlockSpec((2, BLK, 1), lambda i: (0, i, 0))
  row = pl.BlockSpec((BLK, D), lambda i: (i, 0))
  col = pl.BlockSpec((BLK, 1), lambda i: (i, 0))
  full = pl.BlockSpec((D, D), lambda i: (0, 0))
  bias = pl.BlockSpec((1, D), lambda i: (0, 0))
  return pl.pallas_call(
      _tc1_body,
      grid=(NB,),
      in_specs=[row3, col3, row, full, bias, full],
      out_specs=[row, col],
      out_shape=[jax.ShapeDtypeStruct((NPAD, D), f32),
                 jax.ShapeDtypeStruct((NPAD, 1), f32)],
  )(p, cc, x, wl, b, wr)


def _tc2_body(p, inv, h1, qq, w2l, b2, w2r, w3l, b3, w3r,
              out, s1_acc, s2_acc):
  i = pl.program_id(0)
  mean = (p[0, :, :] + p[1, :, :]) * inv[...]
  h2 = jnp.maximum(_dot_t(mean, w2l[...]) + b2[...] + _dot_t(h1[...], w2r[...]),
                   0.0)
  h2 = jnp.where(_row_mask(i), h2, 0.0)
  q = qq[0, :, :] + qq[1, :, :]

  @pl.when(i == 0)
  def _():
    s1_acc[...] = jnp.zeros_like(s1_acc)
    s2_acc[...] = jnp.zeros_like(s2_acc)

  s1_acc[...] += jnp.sum(q * h2, axis=0, keepdims=True)
  s2_acc[...] += jnp.sum(h2, axis=0, keepdims=True)

  @pl.when(i == pl.num_programs(0) - 1)
  def _():
    out[...] = ((_dot_t(s1_acc[...], w3l[...])
                 + _dot_t(s2_acc[...], w3r[...])) * (1.0 / N) + b3[...])


def _tc2(p, inv, h1, qq, w2l, b2, w2r, w3l, b3, w3r):
  row3 = pl.BlockSpec((2, BLK, D), lambda i: (0, i, 0))
  col3 = pl.BlockSpec((2, BLK, 1), lambda i: (0, i, 0))
  row = pl.BlockSpec((BLK, D), lambda i: (i, 0))
  col = pl.BlockSpec((BLK, 1), lambda i: (i, 0))
  full = pl.BlockSpec((D, D), lambda i: (0, 0))
  bias = pl.BlockSpec((1, D), lambda i: (0, 0))
  return pl.pallas_call(
      _tc2_body,
      grid=(NB,),
      in_specs=[row3, col, row, col3,
                full, bias, full, full, bias, full],
      out_specs=pl.BlockSpec((1, D), lambda i: (0, 0)),
      out_shape=jax.ShapeDtypeStruct((1, D), f32),
      scratch_shapes=[pltpu.VMEM((1, D), f32), pltpu.VMEM((1, D), f32)],
      compiler_params=pltpu.CompilerParams(
          dimension_semantics=("arbitrary",)),
  )(p, inv, h1, qq, w2l, b2, w2r, w3l, b3, w3r)


def kernel(x, edge_index, W1l, b1, W1r, W2l, b2, W2r, W3l, b3, W3r):
  ei = edge_index.astype(jnp.int32)
  npad_e = EPTP - EPT
  src = jnp.concatenate(
      [ei[0].reshape(NTILES, EPT),
       jnp.zeros((NTILES, npad_e), jnp.int32)], axis=1).reshape(NTILES, NCH, C)
  dst = jnp.concatenate(
      [ei[1].reshape(NTILES, EPT),
       jnp.full((NTILES, npad_e), NPAD - 1, jnp.int32)],
      axis=1).reshape(NTILES, NCH, C)
  zrows = jnp.zeros((STRIPE, D), f32)
  z1 = jnp.zeros((STRIPE,), f32)
  ones = jnp.ones((C,), f32)
  wdummy = jnp.zeros((NPAD,), f32)

  x_p = jnp.concatenate([x, jnp.zeros((NPAD - N, D), f32)], axis=0)

  p_l1, cnt_p = _seg_cnt(src, dst, x, wdummy, zrows, z1, ones)
  h1, inv = _tc1(p_l1, cnt_p[:, :, None], x_p, W1l, b1[None, :], W1r)
  p_l2, q_p = _seg_q(src, dst, h1, inv[:, 0], zrows, z1, ones)
  out = _tc2(p_l2, inv, h1, q_p[:, :, None],
             W2l, b2[None, :], W2r, W3l, b3[None, :], W3r)
  return out
